# baseline (speedup 1.0000x reference)
import functools

import jax
import jax.numpy as jnp
from jax import lax
from jax.experimental import pallas as pl
from jax.experimental.pallas import tpu as pltpu
from jax.experimental.pallas import tpu_sc as plsc

DIM = 64
NC = 2
NS = 16
NW = NC * NS
BLK = 640
NBUF = 2


def _gather_body(idx_hbm, table_hbm, out_hbm, idx_v, *rest):
    rows = rest[:NBUF]
    gsem = rest[NBUF:2 * NBUF]
    wsem = rest[2 * NBUF:3 * NBUF]

    wid = lax.axis_index("s") * NC + lax.axis_index("c")
    n_per_w = idx_hbm.shape[1]
    nblk = n_per_w // BLK
    base = wid * n_per_w

    pltpu.sync_copy(idx_hbm.at[wid], idx_v)

    def issue_gather(b, s):
        pltpu.async_copy(
            table_hbm.at[idx_v.at[pl.ds(b * BLK, BLK)]],
            rows[s],
            gsem[s])

    def wait_gather(s):
        pltpu.make_async_copy(out_hbm.at[pl.ds(0, BLK)], rows[s],
                              gsem[s]).wait()

    def issue_write(b, s):
        pltpu.async_copy(rows[s], out_hbm.at[pl.ds(base + b * BLK, BLK)],
                         wsem[s])

    def wait_write(s):
        pltpu.make_async_copy(rows[s], out_hbm.at[pl.ds(0, BLK)],
                              wsem[s]).wait()

    for s in range(NBUF):
        issue_gather(s, s)

    def group(q, carry):
        for s in range(NBUF):
            b = q * NBUF + s
            wait_gather(s)
            issue_write(b, s)
            wait_write(s)
            issue_gather(b + NBUF, s)
        return carry

    lax.fori_loop(0, nblk // NBUF - 1, group, 0)

    for s in range(NBUF):
        b = nblk - NBUF + s
        wait_gather(s)
        issue_write(b, s)
    for s in range(NBUF):
        wait_write(s)


def kernel(input, weight):
    B, L = input.shape
    n = B * L
    n_per_w = n // NW
    idx3 = input.reshape(NW, n_per_w)

    mesh = plsc.VectorSubcoreMesh(core_axis_name="c", subcore_axis_name="s")
    scratch = [pltpu.VMEM((n_per_w,), jnp.int32)]
    scratch += [pltpu.VMEM((BLK, DIM), jnp.float32) for _ in range(NBUF)]
    scratch += [pltpu.SemaphoreType.DMA for _ in range(2 * NBUF)]
    run = functools.partial(
        pl.kernel,
        mesh=mesh,
        out_type=jax.ShapeDtypeStruct((n, DIM), jnp.float32),
        scratch_types=scratch,
        compiler_params=pltpu.CompilerParams(use_tc_tiling_on_sc=False),
    )(_gather_body)
    out = run(idx3, weight)
    return out.reshape(B, L, DIM)


# speedup vs baseline: 1.3299x; 1.3299x over previous
import functools

import jax
import jax.numpy as jnp
from jax import lax
from jax.experimental import pallas as pl
from jax.experimental.pallas import tpu as pltpu
from jax.experimental.pallas import tpu_sc as plsc

DIM = 64
PAD = 128
NC = 2
NS = 16
NW = NC * NS
BLK = 256
NBUF = 2


def _gather_body(idx_hbm, table_hbm, out_hbm, idx_v, *rest):
    rows = rest[:NBUF]
    gsem = rest[NBUF:2 * NBUF]
    wsem = rest[2 * NBUF:3 * NBUF]

    wid = lax.axis_index("s") * NC + lax.axis_index("c")
    n_per_w = idx_v.shape[0]
    nblk = n_per_w // BLK
    base = wid * n_per_w

    pltpu.sync_copy(idx_hbm.at[pl.ds(base, n_per_w)], idx_v)

    def issue_gather(b, s):
        pltpu.async_copy(
            table_hbm.at[idx_v.at[pl.ds(b * BLK, BLK)]],
            rows[s],
            gsem[s])

    def wait_gather(s):
        pltpu.make_async_copy(out_hbm.at[pl.ds(0, BLK), pl.ds(0, DIM)],
                              rows[s], gsem[s]).wait()

    def issue_write(b, s):
        pltpu.async_copy(rows[s],
                         out_hbm.at[pl.ds(base + b * BLK, BLK), pl.ds(0, DIM)],
                         wsem[s])

    def wait_write(s):
        pltpu.make_async_copy(rows[s],
                              out_hbm.at[pl.ds(0, BLK), pl.ds(0, DIM)],
                              wsem[s]).wait()

    for s in range(NBUF):
        issue_gather(s, s)

    def group(q, carry):
        for s in range(NBUF):
            b = q * NBUF + s
            wait_gather(s)
            issue_write(b, s)
            wait_write(s)
            issue_gather(b + NBUF, s)
        return carry

    lax.fori_loop(0, nblk // NBUF - 1, group, 0)

    for s in range(NBUF):
        b = nblk - NBUF + s
        wait_gather(s)
        issue_write(b, s)
    for s in range(NBUF):
        wait_write(s)


def kernel(input, weight):
    B, L = input.shape
    V = weight.shape[0]
    n = B * L
    n_per_w = n // NW

    idx_flat = input.reshape(n)

    mesh = plsc.VectorSubcoreMesh(core_axis_name="c", subcore_axis_name="s")
    scratch = [pltpu.VMEM((n_per_w,), jnp.int32)]
    scratch += [pltpu.VMEM((BLK, DIM), jnp.float32) for _ in range(NBUF)]
    scratch += [pltpu.SemaphoreType.DMA for _ in range(2 * NBUF)]
    run = functools.partial(
        pl.kernel,
        mesh=mesh,
        out_type=jax.ShapeDtypeStruct((n, PAD), jnp.float32),
        scratch_types=scratch,
        compiler_params=pltpu.CompilerParams(use_tc_tiling_on_sc=False),
    )(_gather_body)
    out = run(idx_flat, weight)
    return out[:, :DIM].reshape(B, L, DIM)
